# baseline (speedup 1.0000x reference)
import functools

import jax
import jax.numpy as jnp
from jax import lax
from jax.experimental import pallas as pl
from jax.experimental.pallas import tpu as pltpu

_B = 4096
_D = 512
_N = 1000
_R = 512
_G = _B // _R


def _tc_body(labels_ref, feats_ref, ct_ref, out_ref, p_buf, cn_row, col_i):
    i = pl.program_id(0)

    @pl.when(i == 0)
    def _():
        t32 = ct_ref[...].astype(jnp.float32)
        cn_row[...] = 0.25 * jnp.sum(t32 * t32, axis=0, keepdims=True)
        col_i[...] = lax.broadcasted_iota(jnp.int32, (_R, _N), 1)
        out_ref[...] = jnp.zeros((1, 1), jnp.float32)

    @pl.when(i < _G)
    def _():
        F = feats_ref[...]
        f2 = jnp.sum(F * F)
        P = lax.dot_general(
            F.astype(jnp.bfloat16), ct_ref[...],
            (((1,), (0,)), ((), ())), preferred_element_type=jnp.float32)
        p_buf[i % 2] = P
        out_ref[...] += jnp.reshape(f2, (1, 1))

    @pl.when(i > 0)
    def _():
        lab = labels_ref[...]
        mask = col_i[...] == lab
        sel = jnp.sum(jnp.where(mask, cn_row[...] + p_buf[(i + 1) % 2], 0.0))
        out_ref[...] += jnp.reshape(sel, (1, 1))


def kernel(feats, labels, centers):
    lab2 = labels.astype(jnp.int32).reshape(_B, 1)
    ct_bf = (-2.0 * centers.T).astype(jnp.bfloat16)
    out = pl.pallas_call(
        _tc_body,
        grid=(_G + 1,),
        in_specs=[
            pl.BlockSpec((_R, 1), lambda i: (jnp.maximum(i - 1, 0), 0)),
            pl.BlockSpec((_R, _D), lambda i: (jnp.minimum(i, _G - 1), 0)),
            pl.BlockSpec((_D, _N), lambda i: (0, 0)),
        ],
        out_specs=pl.BlockSpec((1, 1), lambda i: (0, 0)),
        out_shape=jax.ShapeDtypeStruct((1, 1), jnp.float32),
        scratch_shapes=[
            pltpu.VMEM((2, _R, _N), jnp.float32),
            pltpu.VMEM((1, _N), jnp.float32),
            pltpu.VMEM((_R, _N), jnp.int32),
        ],
    )(lab2, feats, ct_bf)
    return out[0, 0] / jnp.float32(_B * _D)


# speedup vs baseline: 1.4734x; 1.4734x over previous
import functools

import jax
import jax.numpy as jnp
from jax import lax
from jax.experimental import pallas as pl
from jax.experimental.pallas import tpu as pltpu

_B = 4096
_D = 512
_N = 1000
_R = 512
_G = _B // _R


def _tc_body(labels_ref, feats_ref, centers_ref, out_ref, cb_sc, cn_col, row_i):
    i = pl.program_id(0)

    @pl.when(i == 0)
    def _():
        C = centers_ref[...]
        cb_sc[...] = (-2.0 * C).astype(jnp.bfloat16)
        cn_col[...] = jnp.sum(C * C, axis=1, keepdims=True)
        row_i[...] = lax.broadcasted_iota(jnp.int32, (_N, _R), 0)

    F = feats_ref[...]
    f2 = jnp.sum(F * F)
    Pt = lax.dot_general(
        cb_sc[...], F.astype(jnp.bfloat16),
        (((1,), (1,)), ((), ())), preferred_element_type=jnp.float32)
    lab = labels_ref[...]
    mask = row_i[...] == lab
    sel = jnp.sum(jnp.where(mask, cn_col[...] + Pt, 0.0))
    contrib = jnp.reshape(sel + f2, (1, 1))

    @pl.when(i == 0)
    def _():
        out_ref[...] = contrib

    @pl.when(i > 0)
    def _():
        out_ref[...] += contrib


def kernel(feats, labels, centers):
    lab2 = labels.astype(jnp.int32).reshape(1, _B)
    out = pl.pallas_call(
        _tc_body,
        grid=(_G,),
        in_specs=[
            pl.BlockSpec((1, _R), lambda i: (0, i)),
            pl.BlockSpec((_R, _D), lambda i: (i, 0)),
            pl.BlockSpec((_N, _D), lambda i: (0, 0)),
        ],
        out_specs=pl.BlockSpec((1, 1), lambda i: (0, 0)),
        out_shape=jax.ShapeDtypeStruct((1, 1), jnp.float32),
        scratch_shapes=[
            pltpu.VMEM((_N, _D), jnp.bfloat16),
            pltpu.VMEM((_N, 1), jnp.float32),
            pltpu.VMEM((_N, _R), jnp.int32),
        ],
    )(lab2, feats, centers)
    return out[0, 0] / jnp.float32(_B * _D)


# speedup vs baseline: 1.5694x; 1.0651x over previous
import functools

import jax
import jax.numpy as jnp
from jax import lax
from jax.experimental import pallas as pl
from jax.experimental.pallas import tpu as pltpu

_B = 4096
_D = 512
_N = 1000
_R = 1024
_G = _B // _R


def _tc_body(labels_ref, feats_ref, centers_ref, out_ref, cb_sc, cn_col):
    i = pl.program_id(0)

    @pl.when(i == 0)
    def _():
        C = centers_ref[...]
        cb_sc[...] = (-2.0 * C).astype(jnp.bfloat16)
        cn_col[...] = jnp.sum(C * C, axis=1, keepdims=True)

    F = feats_ref[...]
    f2 = jnp.sum(F * F)
    Pt = lax.dot_general(
        cb_sc[...], F.astype(jnp.bfloat16),
        (((1,), (1,)), ((), ())), preferred_element_type=jnp.float32)
    lab = labels_ref[...]
    row = lax.broadcasted_iota(jnp.int32, (_N, 1), 0)
    mask = row == lab
    sel = jnp.sum(jnp.where(mask, cn_col[...] + Pt, 0.0))
    contrib = jnp.reshape(sel + f2, (1, 1))

    @pl.when(i == 0)
    def _():
        out_ref[...] = contrib

    @pl.when(i > 0)
    def _():
        out_ref[...] += contrib


def kernel(feats, labels, centers):
    lab2 = labels.astype(jnp.int32).reshape(1, _B)
    out = pl.pallas_call(
        _tc_body,
        grid=(_G,),
        in_specs=[
            pl.BlockSpec((1, _R), lambda i: (0, i)),
            pl.BlockSpec((_R, _D), lambda i: (i, 0)),
            pl.BlockSpec((_N, _D), lambda i: (0, 0)),
        ],
        out_specs=pl.BlockSpec((1, 1), lambda i: (0, 0)),
        out_shape=jax.ShapeDtypeStruct((1, 1), jnp.float32),
        scratch_shapes=[
            pltpu.VMEM((_N, _D), jnp.bfloat16),
            pltpu.VMEM((_N, 1), jnp.float32),
        ],
    )(lab2, feats, centers)
    return out[0, 0] / jnp.float32(_B * _D)
